# baseline (speedup 1.0000x reference)
import jax
import jax.numpy as jnp
from jax import lax
from jax.experimental import pallas as pl
from jax.experimental.pallas import tpu as pltpu
from jax.experimental.pallas import tpu_sc as plsc

N_TOKENS = 1000000
D = 32
BATCH = 16384
HIST = 50

NC, NS = 2, 16
NW = NC * NS
B_PER_W = BATCH // NW
BT_PER_W = B_PER_W // 128


def _gather_body(x_hbm, w128_hbm, out_hbm, x2d_v, idx_v, sub_v, rows_v,
                 tiles_v, sem_g, sem_s):
    wid = lax.axis_index("s") * NC + lax.axis_index("c")
    base_b = wid * B_PER_W
    lane = lax.iota(jnp.int32, 16)

    def compact_gather(c, par):
        col = jnp.broadcast_to(c, (16,)).astype(jnp.int32)
        for k in range(8):
            p_vec = k * 16 + lane
            v = plsc.load_gather(x2d_v, [p_vec, col])
            idx_v.at[par][pl.ds(k * 16, 16)] = v >> 2
            sub_v.at[par][pl.ds(k * 16, 16)] = v & 3
        pltpu.async_copy(w128_hbm.at[idx_v.at[par]], rows_v.at[par], sem_g)

    def gather_wait(par):
        pltpu.make_async_copy(w128_hbm.at[idx_v.at[par]], rows_v.at[par],
                              sem_g).wait()

    def assemble(par):
        for bg in range(8):
            p_vec = bg * 16 + lane
            sub16 = plsc.load_gather(sub_v.at[par], [p_vec])
            colb = sub16 * 32
            for dt in range(4):
                for d8 in range(8):
                    val = plsc.load_gather(rows_v.at[par],
                                           [p_vec, colb + (dt * 8 + d8)])
                    tiles_v.at[par][dt, d8, pl.ds(bg * 16, 16)] = val

    def store_start(c, bt, par):
        gb0 = base_b + bt * 128
        for dt in range(4):
            pltpu.async_copy(tiles_v.at[par].at[dt],
                             out_hbm.at[c, pl.ds(dt * 8, 8), pl.ds(gb0, 128)],
                             sem_s)

    def store_wait(c, bt, par):
        gb0 = base_b + bt * 128
        for dt in range(4):
            pltpu.make_async_copy(
                tiles_v.at[par].at[dt],
                out_hbm.at[c, pl.ds(dt * 8, 8), pl.ds(gb0, 128)],
                sem_s).wait()

    @pl.loop(0, BT_PER_W)
    def _btile(bt):
        pltpu.sync_copy(x_hbm.at[pl.ds(base_b + bt * 128, 128)], x2d_v)

        for par in range(2):
            compact_gather(par, par)

        @pl.loop(0, HIST - 2, step=2)
        def _steady(c0):
            for par in range(2):
                c = c0 + par
                gather_wait(par)

                @pl.when(c0 >= 2)
                def _():
                    store_wait(c - 2, bt, par)

                assemble(par)
                store_start(c, bt, par)
                compact_gather(c + 2, par)

        for par in range(2):
            c = HIST - 2 + par
            gather_wait(par)
            store_wait(c - 2, bt, par)
            assemble(par)
            store_start(c, bt, par)
        for par in range(2):
            store_wait(HIST - 2 + par, bt, par)


def kernel(x, weights):
    x32 = x.astype(jnp.int32)
    mesh = plsc.VectorSubcoreMesh(core_axis_name="c", subcore_axis_name="s",
                                  num_cores=NC, num_subcores=NS)
    w128 = weights.reshape(N_TOKENS // 4, 128)
    out5 = pl.kernel(
        _gather_body,
        out_type=jax.ShapeDtypeStruct((HIST, D, BATCH), jnp.float32),
        mesh=mesh,
        scratch_types=[
            pltpu.VMEM((128, HIST), jnp.int32),
            pltpu.VMEM((2, 128), jnp.int32),
            pltpu.VMEM((2, 128), jnp.int32),
            pltpu.VMEM((2, 128, 128), jnp.float32),
            pltpu.VMEM((2, 4, 8, 128), jnp.float32),
            pltpu.SemaphoreType.DMA,
            pltpu.SemaphoreType.DMA,
        ],
        compiler_params=pltpu.CompilerParams(use_tc_tiling_on_sc=True,
                                             needs_layout_passes=False),
    )(x32, w128)
    return out5.transpose(2, 0, 1)


# speedup vs baseline: 1.0170x; 1.0170x over previous
import jax
import jax.numpy as jnp
from jax import lax
from jax.experimental import pallas as pl
from jax.experimental.pallas import tpu as pltpu
from jax.experimental.pallas import tpu_sc as plsc

N_TOKENS = 1000000
D = 32
BATCH = 16384
HIST = 50

NC, NS = 2, 16
NW = NC * NS
B_PER_W = BATCH // NW
BT_PER_W = B_PER_W // 128
NSUB = BT_PER_W * HIST
NBUF = 3


def _gather_body(x_hbm, w128_hbm, out_hbm, x2d_v, idx_v, sub_v, rows_v,
                 tiles_v, sem_g, sem_s):
    wid = lax.axis_index("s") * NC + lax.axis_index("c")
    base_b = wid * B_PER_W
    lane = lax.iota(jnp.int32, 16)

    def compact_gather(c, par):
        bt = c // HIST
        h = c - bt * HIST

        @pl.when(h == 0)
        def _():
            pltpu.sync_copy(x_hbm.at[pl.ds(base_b + bt * 128, 128)], x2d_v)

        col = jnp.broadcast_to(h, (16,)).astype(jnp.int32)
        for k in range(8):
            p_vec = k * 16 + lane
            v = plsc.load_gather(x2d_v, [p_vec, col])
            idx_v.at[par][pl.ds(k * 16, 16)] = v >> 2
            sub_v.at[par][pl.ds(k * 16, 16)] = v & 3
        pltpu.async_copy(w128_hbm.at[idx_v.at[par]], rows_v.at[par], sem_g)

    def gather_wait(par):
        pltpu.make_async_copy(w128_hbm.at[idx_v.at[par]], rows_v.at[par],
                              sem_g).wait()

    def assemble(par):
        for bg in range(8):
            p_vec = bg * 16 + lane
            sub16 = plsc.load_gather(sub_v.at[par], [p_vec])
            colb = sub16 * 32
            for dt in range(4):
                for d8 in range(8):
                    val = plsc.load_gather(rows_v.at[par],
                                           [p_vec, colb + (dt * 8 + d8)])
                    tiles_v.at[par][dt, d8, pl.ds(bg * 16, 16)] = val

    def store_start(c, par):
        bt = c // HIST
        h = c - bt * HIST
        gb0 = base_b + bt * 128
        for dt in range(4):
            pltpu.async_copy(tiles_v.at[par].at[dt],
                             out_hbm.at[h, pl.ds(dt * 8, 8), pl.ds(gb0, 128)],
                             sem_s)

    def store_wait(c, par):
        bt = c // HIST
        h = c - bt * HIST
        gb0 = base_b + bt * 128
        for dt in range(4):
            pltpu.make_async_copy(
                tiles_v.at[par].at[dt],
                out_hbm.at[h, pl.ds(dt * 8, 8), pl.ds(gb0, 128)],
                sem_s).wait()

    for par in range(NBUF):
        compact_gather(par, par)

    @pl.loop(0, NSUB - 2, step=NBUF)
    def _steady(c0):
        for parn in range(NBUF):
            c = c0 + parn
            gather_wait(parn)

            @pl.when(c0 >= NBUF)
            def _():
                store_wait(c - NBUF, parn)

            assemble(parn)
            store_start(c, parn)

            @pl.when(c + NBUF < NSUB)
            def _():
                compact_gather(c + NBUF, parn)

    for parn in range(2):
        c = NSUB - 2 + parn
        gather_wait(parn)
        store_wait(c - NBUF, parn)
        assemble(parn)
        store_start(c, parn)
    for c in range(NSUB - NBUF, NSUB):
        store_wait(c, c % NBUF)


def kernel(x, weights):
    x32 = x.astype(jnp.int32)
    mesh = plsc.VectorSubcoreMesh(core_axis_name="c", subcore_axis_name="s",
                                  num_cores=NC, num_subcores=NS)
    w128 = weights.reshape(N_TOKENS // 4, 128)
    out5 = pl.kernel(
        _gather_body,
        out_type=jax.ShapeDtypeStruct((HIST, D, BATCH), jnp.float32),
        mesh=mesh,
        scratch_types=[
            pltpu.VMEM((128, HIST), jnp.int32),
            pltpu.VMEM((NBUF, 128), jnp.int32),
            pltpu.VMEM((NBUF, 128), jnp.int32),
            pltpu.VMEM((NBUF, 128, 128), jnp.float32),
            pltpu.VMEM((NBUF, 4, 8, 128), jnp.float32),
            pltpu.SemaphoreType.DMA,
            pltpu.SemaphoreType.DMA,
        ],
        compiler_params=pltpu.CompilerParams(use_tc_tiling_on_sc=True,
                                             needs_layout_passes=False),
    )(x32, w128)
    return out5.transpose(2, 0, 1)


# speedup vs baseline: 1.1667x; 1.1472x over previous
import jax
import jax.numpy as jnp
from jax import lax
from jax.experimental import pallas as pl
from jax.experimental.pallas import tpu as pltpu
from jax.experimental.pallas import tpu_sc as plsc

N_TOKENS = 1000000
D = 32
BATCH = 16384
HIST = 50

NC, NS = 2, 16
NW = NC * NS
B_PER_W = BATCH // NW
RB = 32
CHUNK = RB * HIST
NCHUNK = B_PER_W // RB
NBUF = 2
NVEC = CHUNK // 16


def _gather_body(x_hbm, w_hbm, out_hbm, x2d_v, idx_v, rows_v, sem_g, sem_s):
    wid = lax.axis_index("s") * NC + lax.axis_index("c")
    base = wid * B_PER_W
    lane = lax.iota(jnp.int32, 16)

    def idx_gather_start(c, b):
        off = base + c * RB
        pltpu.sync_copy(x_hbm.at[pl.ds(off, RB)], x2d_v.at[b])

        @pl.loop(0, NVEC)
        def _compact(k):
            flat = k * 16 + lane
            r = flat // HIST
            col = flat - r * HIST
            v = plsc.load_gather(x2d_v.at[b], [r, col])
            idx_v.at[b][pl.ds(k * 16, 16)] = v

        pltpu.async_copy(w_hbm.at[idx_v.at[b]], rows_v.at[b], sem_g)

    def gather_wait(b):
        pltpu.make_async_copy(w_hbm.at[idx_v.at[b]], rows_v.at[b], sem_g).wait()

    def store_start(c, b):
        off = base + c * RB

        @pl.loop(0, RB)
        def _rows(r):
            pltpu.async_copy(rows_v.at[b].at[pl.ds(r * HIST, HIST)],
                             out_hbm.at[off + r], sem_s)

    def store_wait(c, b):
        off = base + c * RB

        @pl.loop(0, RB)
        def _rows(r):
            pltpu.make_async_copy(rows_v.at[b].at[pl.ds(r * HIST, HIST)],
                                  out_hbm.at[off + r], sem_s).wait()

    for b in range(NBUF):
        idx_gather_start(b, b)

    @pl.loop(0, NCHUNK - NBUF, step=NBUF)
    def _steady(c0):
        for b in range(NBUF):
            c = c0 + b
            gather_wait(b)
            store_start(c, b)
            store_wait(c, b)
            idx_gather_start(c + NBUF, b)

    for b in range(NBUF):
        gather_wait(b)
        store_start(NCHUNK - NBUF + b, b)
    for b in range(NBUF):
        store_wait(NCHUNK - NBUF + b, b)


def kernel(x, weights):
    x32 = x.astype(jnp.int32)
    mesh = plsc.VectorSubcoreMesh(core_axis_name="c", subcore_axis_name="s",
                                  num_cores=NC, num_subcores=NS)
    out = pl.kernel(
        _gather_body,
        out_type=jax.ShapeDtypeStruct((BATCH, HIST, D), jnp.float32),
        mesh=mesh,
        scratch_types=[
            pltpu.VMEM((NBUF, RB, HIST), jnp.int32),
            pltpu.VMEM((NBUF, CHUNK), jnp.int32),
            pltpu.VMEM((NBUF, CHUNK, D), jnp.float32),
            pltpu.SemaphoreType.DMA,
            pltpu.SemaphoreType.DMA,
        ],
        compiler_params=pltpu.CompilerParams(use_tc_tiling_on_sc=False,
                                             needs_layout_passes=False),
    )(x32, weights)
    return out


# speedup vs baseline: 1.1691x; 1.0021x over previous
import jax
import jax.numpy as jnp
from jax import lax
from jax.experimental import pallas as pl
from jax.experimental.pallas import tpu as pltpu
from jax.experimental.pallas import tpu_sc as plsc

N_TOKENS = 1000000
D = 32
BATCH = 16384
HIST = 50

NC, NS = 2, 16
NW = NC * NS
B_PER_W = BATCH // NW
RB = 32
CHUNK = RB * HIST
NCHUNK = B_PER_W // RB
NBUF = 2
NVEC = CHUNK // 16


def _gather_body(x_hbm, w_hbm, out_hbm, x2d_v, idx_v, rows_v, sem_g, sem_s):
    wid = lax.axis_index("s") * NC + lax.axis_index("c")
    base = wid * B_PER_W
    lane = lax.iota(jnp.int32, 16)
    zero = lane * 0

    def idx_gather_start(c, b):
        row = wid * NCHUNK + c
        pltpu.sync_copy(x_hbm.at[pl.ds(row, 1)], x2d_v.at[b])

        @pl.loop(0, NVEC)
        def _compact(k):
            flat = k * 16 + lane
            v = plsc.load_gather(x2d_v.at[b], [zero, flat])
            idx_v.at[b][pl.ds(k * 16, 16)] = v

        pltpu.async_copy(w_hbm.at[idx_v.at[b]], rows_v.at[b], sem_g)

    def gather_wait(b):
        pltpu.make_async_copy(w_hbm.at[idx_v.at[b]], rows_v.at[b], sem_g).wait()

    def store_start(c, b):
        off = base + c * RB

        @pl.loop(0, RB)
        def _rows(r):
            pltpu.async_copy(rows_v.at[b].at[pl.ds(r * HIST, HIST)],
                             out_hbm.at[off + r], sem_s)

    def store_wait(c, b):
        off = base + c * RB

        @pl.loop(0, RB)
        def _rows(r):
            pltpu.make_async_copy(rows_v.at[b].at[pl.ds(r * HIST, HIST)],
                                  out_hbm.at[off + r], sem_s).wait()

    for b in range(NBUF):
        idx_gather_start(b, b)

    @pl.loop(0, NCHUNK - NBUF, step=NBUF)
    def _steady(c0):
        for b in range(NBUF):
            c = c0 + b
            gather_wait(b)
            store_start(c, b)
            store_wait(c, b)
            idx_gather_start(c + NBUF, b)

    for b in range(NBUF):
        gather_wait(b)
        store_start(NCHUNK - NBUF + b, b)
    for b in range(NBUF):
        store_wait(NCHUNK - NBUF + b, b)


def kernel(x, weights):
    x32 = x.astype(jnp.int32)
    mesh = plsc.VectorSubcoreMesh(core_axis_name="c", subcore_axis_name="s",
                                  num_cores=NC, num_subcores=NS)
    out = pl.kernel(
        _gather_body,
        out_type=jax.ShapeDtypeStruct((BATCH, HIST, D), jnp.float32),
        mesh=mesh,
        scratch_types=[
            pltpu.VMEM((NBUF, 1, CHUNK), jnp.int32),
            pltpu.VMEM((NBUF, CHUNK), jnp.int32),
            pltpu.VMEM((NBUF, CHUNK, D), jnp.float32),
            pltpu.SemaphoreType.DMA,
            pltpu.SemaphoreType.DMA,
        ],
        compiler_params=pltpu.CompilerParams(use_tc_tiling_on_sc=False,
                                             needs_layout_passes=False),
    )(x32.reshape(BATCH // RB, CHUNK), weights)
    return out
